# baseline (speedup 1.0000x reference)
import functools

import jax
import jax.numpy as jnp
from jax import lax
from jax.experimental import pallas as pl
from jax.experimental.pallas import tpu as pltpu
from jax.experimental.pallas import tpu_sc as plsc

NC = 2
NS = 16
NW = NC * NS
CH = 40
DW = 16


def _sc_mesh():
  return plsc.VectorSubcoreMesh(
      core_axis_name="c", subcore_axis_name="s", num_cores=NC, num_subcores=NS)


def _sc_degree(dstr, npad):
  nrnd = dstr.shape[1]
  rpt = npad // NS
  zr = 128

  @functools.partial(
      pl.kernel,
      out_type=jax.ShapeDtypeStruct((NC, npad, DW), jnp.float32),
      mesh=_sc_mesh(),
      scratch_types=[
          pltpu.VMEM((2, _RING, CH), jnp.int32),
          pltpu.VMEM((CH, DW), jnp.float32),
          pltpu.VMEM((zr, DW), jnp.float32),
          pltpu.VMEM_SHARED((npad, DW), jnp.float32),
          pltpu.SemaphoreType.DMA,
          pltpu.SemaphoreType.DMA((2,)),
      ],
  )
  def k(dst_hbm, out_hbm, didx, ones, zbuf, acc, dsem, isem):
    c = lax.axis_index("c")
    s = lax.axis_index("s")
    w = c * NS + s

    pltpu.async_copy(dst_hbm.at[w, 0], didx.at[0], isem.at[0])
    pltpu.async_copy(dst_hbm.at[w, 1], didx.at[1], isem.at[1])

    def fill_z(r, _):
      zbuf[r, :] = jnp.zeros((DW,), jnp.float32)
      return 0
    lax.fori_loop(0, zr, fill_z, 0)

    def fill_o(r, _):
      ones[r, :] = jnp.ones((DW,), jnp.float32)
      return 0
    lax.fori_loop(0, CH, fill_o, 0)

    for z in range(rpt // zr):
      pltpu.async_copy(zbuf, acc.at[pl.ds(s * rpt + z * zr, zr)], dsem)
    for z in range(rpt // zr):
      pltpu.make_async_copy(zbuf, acc.at[pl.ds(s * rpt, zr)], dsem).wait()
    plsc.subcore_barrier()

    def do_round(m, p):
      for b in range(_RING):
        pltpu.async_copy(ones, acc.at[didx.at[p].at[b]], dsem, add=True)
      for b in range(_RING):
        pltpu.make_async_copy(ones, acc.at[didx.at[p].at[b]], dsem).wait()

    def dround(t, _):
      for p in range(2):
        m = 2 * t + p
        pltpu.make_async_copy(dst_hbm.at[w, m], didx.at[p], isem.at[p]).wait()
        do_round(m, p)

        @pl.when(m + 2 < nrnd)
        def _():
          pltpu.async_copy(dst_hbm.at[w, m + 2], didx.at[p], isem.at[p])
      return 0
    lax.fori_loop(0, nrnd // 2, dround, 0)
    if nrnd % 2:
      m = nrnd - 1
      pltpu.make_async_copy(dst_hbm.at[w, m], didx.at[0], isem.at[0]).wait()
      do_round(m, 0)
    plsc.subcore_barrier()

    pltpu.sync_copy(acc.at[pl.ds(s * rpt, rpt)],
                    out_hbm.at[c, pl.ds(s * rpt, rpt)])

  return k(dstr)


_RING = 8


def _sc_aggregate(y, srcr, dstr, npad):
  n, d = y.shape
  nrnd = srcr.shape[1]
  rpt = npad // NS
  assert rpt % CH == 0

  @functools.partial(
      pl.kernel,
      out_type=jax.ShapeDtypeStruct((NC, npad, d), jnp.float32),
      mesh=_sc_mesh(),
      scratch_types=[
          pltpu.VMEM((2, _RING, CH), jnp.int32),
          pltpu.VMEM((2, _RING, CH), jnp.int32),
          pltpu.VMEM((_RING, CH, d), jnp.float32),
          pltpu.VMEM_SHARED((npad, d), jnp.float32),
          pltpu.SemaphoreType.DMA((_RING,)),
          pltpu.SemaphoreType.DMA((2,)),
      ],
  )
  def k(y_hbm, src_hbm, dst_hbm, out_hbm, sidx, didx, rows, acc, gsem, isem):
    c = lax.axis_index("c")
    s = lax.axis_index("s")
    w = c * NS + s

    for p in range(2):
      pltpu.async_copy(src_hbm.at[w, p], sidx.at[p], isem.at[p])
      pltpu.async_copy(dst_hbm.at[w, p], didx.at[p], isem.at[p])

    def fill_z(r, _):
      for j in range(d // 16):
        rows[0, r, pl.ds(j * 16, 16)] = jnp.zeros((16,), jnp.float32)
      return 0
    lax.fori_loop(0, CH, fill_z, 0)
    for z in range(rpt // CH):
      pltpu.async_copy(rows.at[0], acc.at[pl.ds(s * rpt + z * CH, CH)],
                       gsem.at[0])
    for z in range(rpt // CH):
      pltpu.make_async_copy(rows.at[0], acc.at[pl.ds(s * rpt, CH)],
                            gsem.at[0]).wait()
    plsc.subcore_barrier()

    def wait_idx(m, p):
      pltpu.make_async_copy(src_hbm.at[w, m], sidx.at[p], isem.at[p]).wait()
      pltpu.make_async_copy(dst_hbm.at[w, m], didx.at[p], isem.at[p]).wait()

    def gather(p, b):
      pltpu.async_copy(y_hbm.at[sidx.at[p].at[b]], rows.at[b], gsem.at[b])

    def wait_gather(p, b):
      pltpu.make_async_copy(y_hbm.at[sidx.at[p].at[b]], rows.at[b],
                            gsem.at[b]).wait()

    wait_idx(0, 0)
    for b in range(_RING):
      gather(0, b)

    def dround(t, _):
      for p in range(2):
        m = 2 * t + p
        last = m + 1 >= nrnd

        @pl.when(jnp.logical_not(last))
        def _():
          wait_idx(m + 1, 1 - p)
        for b in range(_RING):
          wait_gather(p, b)
          pltpu.sync_copy(rows.at[b], acc.at[didx.at[p].at[b]], add=True)

          @pl.when(jnp.logical_not(last))
          def _():
            gather(1 - p, b)

        @pl.when(m + 2 < nrnd)
        def _():
          pltpu.async_copy(src_hbm.at[w, m + 2], sidx.at[p], isem.at[p])
          pltpu.async_copy(dst_hbm.at[w, m + 2], didx.at[p], isem.at[p])
      return 0
    lax.fori_loop(0, nrnd // 2, dround, 0)
    assert nrnd % 2 == 0
    plsc.subcore_barrier()

    pltpu.sync_copy(acc.at[pl.ds(s * rpt, rpt)],
                    out_hbm.at[c, pl.ds(s * rpt, rpt)])

  return k(y, srcr, dstr)


_ROWS = 2000


def _tc_first(degp, x, w1):
  n, d = x.shape
  grid = n // _ROWS

  def body(deg_a, deg_b, x_ref, w_ref, y_ref, dinv_ref):
    dg = 1.0 + deg_a[0, :, 0:1] + deg_b[0, :, 0:1]
    dinv = lax.rsqrt(dg)
    y_ref[...] = dinv * jnp.dot(x_ref[...], w_ref[...],
                                preferred_element_type=jnp.float32)
    dinv_ref[...] = dinv

  return pl.pallas_call(
      body,
      grid=(grid,),
      in_specs=[
          pl.BlockSpec((1, _ROWS, DW), lambda i: (0, i, 0)),
          pl.BlockSpec((1, _ROWS, DW), lambda i: (1, i, 0)),
          pl.BlockSpec((_ROWS, d), lambda i: (i, 0)),
          pl.BlockSpec((d, d), lambda i: (0, 0)),
      ],
      out_specs=[
          pl.BlockSpec((_ROWS, d), lambda i: (i, 0)),
          pl.BlockSpec((_ROWS, 1), lambda i: (i, 0)),
      ],
      out_shape=[
          jax.ShapeDtypeStruct((n, d), jnp.float32),
          jax.ShapeDtypeStruct((n, 1), jnp.float32),
      ],
  )(degp, degp, x, w1)


def _tc_mid(agg, y, dinv, b, w):
  n, d = y.shape
  grid = n // _ROWS

  def body(agg_a, agg_b, y_ref, dinv_ref, b_ref, w_ref, out_ref):
    dv = dinv_ref[...]
    h = jnp.maximum(
        dv * (agg_a[0] + agg_b[0] + y_ref[...]) + b_ref[...], 0.0)
    out_ref[...] = dv * jnp.dot(h, w_ref[...],
                                preferred_element_type=jnp.float32)

  return pl.pallas_call(
      body,
      grid=(grid,),
      in_specs=[
          pl.BlockSpec((1, _ROWS, d), lambda i: (0, i, 0)),
          pl.BlockSpec((1, _ROWS, d), lambda i: (1, i, 0)),
          pl.BlockSpec((_ROWS, d), lambda i: (i, 0)),
          pl.BlockSpec((_ROWS, 1), lambda i: (i, 0)),
          pl.BlockSpec((1, d), lambda i: (0, 0)),
          pl.BlockSpec((d, d), lambda i: (0, 0)),
      ],
      out_specs=pl.BlockSpec((_ROWS, d), lambda i: (i, 0)),
      out_shape=jax.ShapeDtypeStruct((n, d), jnp.float32),
  )(agg, agg, y, dinv, b.reshape(1, d), w)


def _tc_final(agg, y, dinv, b, batch, wl, bl, g=64):
  n, d = y.shape
  c = wl.shape[1]
  grid = n // _ROWS
  batch_r = batch.reshape(grid, 1, _ROWS)

  def body(agg_a, agg_b, y_ref, dinv_ref, b_ref, batch_ref, wl_ref, bl_ref,
           out_ref, acc_ref):
    i = pl.program_id(0)
    dv = dinv_ref[...]
    h = jnp.maximum(
        dv * (agg_a[0] + agg_b[0] + y_ref[...]) + b_ref[...], 0.0)
    ids = batch_ref[0, 0, :]
    oh = (lax.broadcasted_iota(jnp.int32, (g, _ROWS), 0)
          == jnp.reshape(ids, (1, _ROWS))).astype(jnp.float32)
    part = jnp.dot(oh, h, preferred_element_type=jnp.float32)

    @pl.when(i == 0)
    def _():
      acc_ref[...] = part

    @pl.when(i > 0)
    def _():
      acc_ref[...] = acc_ref[...] + part

    @pl.when(i == grid - 1)
    def _():
      logits = jnp.dot(acc_ref[...], wl_ref[...],
                       preferred_element_type=jnp.float32) + bl_ref[...]
      m = jnp.max(logits, axis=1, keepdims=True)
      lse = jnp.log(jnp.sum(jnp.exp(logits - m), axis=1, keepdims=True)) + m
      out_ref[...] = logits - lse

  return pl.pallas_call(
      body,
      grid=(grid,),
      in_specs=[
          pl.BlockSpec((1, _ROWS, d), lambda i: (0, i, 0)),
          pl.BlockSpec((1, _ROWS, d), lambda i: (1, i, 0)),
          pl.BlockSpec((_ROWS, d), lambda i: (i, 0)),
          pl.BlockSpec((_ROWS, 1), lambda i: (i, 0)),
          pl.BlockSpec((1, d), lambda i: (0, 0)),
          pl.BlockSpec((1, 1, _ROWS), lambda i: (i, 0, 0)),
          pl.BlockSpec((d, c), lambda i: (0, 0)),
          pl.BlockSpec((1, c), lambda i: (0, 0)),
      ],
      out_specs=pl.BlockSpec((g, c), lambda i: (0, 0)),
      out_shape=jax.ShapeDtypeStruct((g, c), jnp.float32),
      scratch_shapes=[pltpu.VMEM((g, d), jnp.float32)],
  )(agg, agg, y, dinv, b.reshape(1, d), batch_r, wl, bl.reshape(1, c))


def kernel(x, edge_index, batch, W1, b1, W2, b2, W3, b3, Wl, bl):
  n, _ = x.shape
  npad = ((n + NS * CH - 1) // (NS * CH)) * NS * CH
  e = edge_index.shape[1]
  gr = 2 * NW * _RING * CH
  ep = ((e + gr - 1) // gr) * gr
  nrnd = ep // (NW * _RING * CH)
  src = jnp.concatenate(
      [edge_index[0], jnp.zeros((ep - e,), edge_index.dtype)])
  dst = jnp.concatenate(
      [edge_index[1], jnp.full((ep - e,), npad - 1, edge_index.dtype)])
  srcr = src.reshape(NW, nrnd, _RING, CH)
  dstr = dst.reshape(NW, nrnd, _RING, CH)

  degp = _sc_degree(dstr, npad)
  y1, dinv = _tc_first(degp, x, W1)
  p1 = _sc_aggregate(y1, srcr, dstr, npad)
  y2 = _tc_mid(p1, y1, dinv, b1, W2)
  p2 = _sc_aggregate(y2, srcr, dstr, npad)
  y3 = _tc_mid(p2, y2, dinv, b2, W3)
  p3 = _sc_aggregate(y3, srcr, dstr, npad)
  return _tc_final(p3, y3, dinv, b3, batch, Wl, bl)


# speedup vs baseline: 2.1471x; 2.1471x over previous
import functools

import jax
import jax.numpy as jnp
from jax import lax
from jax.experimental import pallas as pl
from jax.experimental.pallas import tpu as pltpu
from jax.experimental.pallas import tpu_sc as plsc

NC = 2
NS = 16
NW = NC * NS
CH = 40
DW = 16


def _sc_mesh():
  return plsc.VectorSubcoreMesh(
      core_axis_name="c", subcore_axis_name="s", num_cores=NC, num_subcores=NS)


def _sc_degree(dstr, npad):
  nrnd = dstr.shape[1]
  rpt = npad // NS
  zr = 128

  @functools.partial(
      pl.kernel,
      out_type=jax.ShapeDtypeStruct((NC, npad, DW), jnp.float32),
      mesh=_sc_mesh(),
      scratch_types=[
          pltpu.VMEM((2, _RING, CH), jnp.int32),
          pltpu.VMEM((CH, DW), jnp.float32),
          pltpu.VMEM((zr, DW), jnp.float32),
          pltpu.VMEM_SHARED((npad, DW), jnp.float32),
          pltpu.SemaphoreType.DMA,
          pltpu.SemaphoreType.DMA((2,)),
      ],
  )
  def k(dst_hbm, out_hbm, didx, ones, zbuf, acc, dsem, isem):
    c = lax.axis_index("c")
    s = lax.axis_index("s")
    w = c * NS + s

    pltpu.async_copy(dst_hbm.at[w, 0], didx.at[0], isem.at[0])
    pltpu.async_copy(dst_hbm.at[w, 1], didx.at[1], isem.at[1])

    def fill_z(r, _):
      zbuf[r, :] = jnp.zeros((DW,), jnp.float32)
      return 0
    lax.fori_loop(0, zr, fill_z, 0)

    def fill_o(r, _):
      ones[r, :] = jnp.ones((DW,), jnp.float32)
      return 0
    lax.fori_loop(0, CH, fill_o, 0)

    for z in range(rpt // zr):
      pltpu.async_copy(zbuf, acc.at[pl.ds(s * rpt + z * zr, zr)], dsem)
    for z in range(rpt // zr):
      pltpu.make_async_copy(zbuf, acc.at[pl.ds(s * rpt, zr)], dsem).wait()
    plsc.subcore_barrier()

    def do_round(m, p):
      for b in range(_RING):
        pltpu.async_copy(ones, acc.at[didx.at[p].at[b]], dsem, add=True)
      for b in range(_RING):
        pltpu.make_async_copy(ones, acc.at[didx.at[p].at[b]], dsem).wait()

    def dround(t, _):
      for p in range(2):
        m = 2 * t + p
        pltpu.make_async_copy(dst_hbm.at[w, m], didx.at[p], isem.at[p]).wait()
        do_round(m, p)

        @pl.when(m + 2 < nrnd)
        def _():
          pltpu.async_copy(dst_hbm.at[w, m + 2], didx.at[p], isem.at[p])
      return 0
    lax.fori_loop(0, nrnd // 2, dround, 0)
    if nrnd % 2:
      m = nrnd - 1
      pltpu.make_async_copy(dst_hbm.at[w, m], didx.at[0], isem.at[0]).wait()
      do_round(m, 0)
    plsc.subcore_barrier()

    pltpu.sync_copy(acc.at[pl.ds(s * rpt, rpt)],
                    out_hbm.at[c, pl.ds(s * rpt, rpt)])

  return k(dstr)


_RING = 6


def _sc_aggregate(y, srcr, dstr, npad):
  n, d = y.shape
  nrnd = srcr.shape[1]
  rpt = npad // NS
  assert rpt % CH == 0

  @functools.partial(
      pl.kernel,
      out_type=jax.ShapeDtypeStruct((NC, npad, d), jnp.float32),
      mesh=_sc_mesh(),
      scratch_types=[
          pltpu.VMEM((2, _RING, CH), jnp.int32),
          pltpu.VMEM((2, _RING, CH), jnp.int32),
          pltpu.VMEM((_RING, CH, d), jnp.float32),
          pltpu.VMEM_SHARED((npad, d), jnp.float32),
          pltpu.SemaphoreType.DMA((_RING,)),
          pltpu.SemaphoreType.DMA((2,)),
      ],
  )
  def k(y_hbm, src_hbm, dst_hbm, out_hbm, sidx, didx, rows, acc, gsem, isem):
    c = lax.axis_index("c")
    s = lax.axis_index("s")
    w = c * NS + s

    for p in range(2):
      pltpu.async_copy(src_hbm.at[w, p], sidx.at[p], isem.at[p])
      pltpu.async_copy(dst_hbm.at[w, p], didx.at[p], isem.at[p])

    def fill_z(r, _):
      for j in range(d // 16):
        rows[0, r, pl.ds(j * 16, 16)] = jnp.zeros((16,), jnp.float32)
      return 0
    lax.fori_loop(0, CH, fill_z, 0)
    for z in range(rpt // CH):
      pltpu.async_copy(rows.at[0], acc.at[pl.ds(s * rpt + z * CH, CH)],
                       gsem.at[0])
    for z in range(rpt // CH):
      pltpu.make_async_copy(rows.at[0], acc.at[pl.ds(s * rpt, CH)],
                            gsem.at[0]).wait()
    plsc.subcore_barrier()

    def wait_idx(m, p):
      pltpu.make_async_copy(src_hbm.at[w, m], sidx.at[p], isem.at[p]).wait()
      pltpu.make_async_copy(dst_hbm.at[w, m], didx.at[p], isem.at[p]).wait()

    def gather(p, b):
      pltpu.async_copy(y_hbm.at[sidx.at[p].at[b]], rows.at[b], gsem.at[b])

    def wait_gather(p, b):
      pltpu.make_async_copy(y_hbm.at[sidx.at[p].at[b]], rows.at[b],
                            gsem.at[b]).wait()

    wait_idx(0, 0)
    for b in range(_RING):
      gather(0, b)

    def dround(t, _):
      for p in range(2):
        m = 2 * t + p
        last = m + 1 >= nrnd

        @pl.when(jnp.logical_not(last))
        def _():
          wait_idx(m + 1, 1 - p)
        for b in range(_RING):
          wait_gather(p, b)
          pltpu.sync_copy(rows.at[b], acc.at[didx.at[p].at[b]], add=True)

          @pl.when(jnp.logical_not(last))
          def _():
            gather(1 - p, b)

        @pl.when(m + 2 < nrnd)
        def _():
          pltpu.async_copy(src_hbm.at[w, m + 2], sidx.at[p], isem.at[p])
          pltpu.async_copy(dst_hbm.at[w, m + 2], didx.at[p], isem.at[p])
      return 0
    lax.fori_loop(0, nrnd // 2, dround, 0)
    assert nrnd % 2 == 0
    plsc.subcore_barrier()

    pltpu.sync_copy(acc.at[pl.ds(s * rpt, rpt)],
                    out_hbm.at[c, pl.ds(s * rpt, rpt)])

  return k(y, srcr, dstr)


_ROWS = 2000


def _tc_first(degp, x, w1):
  n, d = x.shape
  grid = n // _ROWS

  def body(deg_a, deg_b, x_ref, w_ref, y_ref, dinv_ref):
    dg = 1.0 + deg_a[0, :, 0:1] + deg_b[0, :, 0:1]
    dinv = lax.rsqrt(dg)
    y_ref[...] = dinv * jnp.dot(x_ref[...], w_ref[...],
                                preferred_element_type=jnp.float32)
    dinv_ref[...] = dinv

  return pl.pallas_call(
      body,
      grid=(grid,),
      in_specs=[
          pl.BlockSpec((1, _ROWS, DW), lambda i: (0, i, 0)),
          pl.BlockSpec((1, _ROWS, DW), lambda i: (1, i, 0)),
          pl.BlockSpec((_ROWS, d), lambda i: (i, 0)),
          pl.BlockSpec((d, d), lambda i: (0, 0)),
      ],
      out_specs=[
          pl.BlockSpec((_ROWS, d), lambda i: (i, 0)),
          pl.BlockSpec((_ROWS, 1), lambda i: (i, 0)),
      ],
      out_shape=[
          jax.ShapeDtypeStruct((n, d), jnp.float32),
          jax.ShapeDtypeStruct((n, 1), jnp.float32),
      ],
  )(degp, degp, x, w1)


def _tc_mid(agg, y, dinv, b, w):
  n, d = y.shape
  grid = n // _ROWS

  def body(agg_a, agg_b, y_ref, dinv_ref, b_ref, w_ref, out_ref):
    dv = dinv_ref[...]
    h = jnp.maximum(
        dv * (agg_a[0] + agg_b[0] + y_ref[...]) + b_ref[...], 0.0)
    out_ref[...] = dv * jnp.dot(h, w_ref[...],
                                preferred_element_type=jnp.float32)

  return pl.pallas_call(
      body,
      grid=(grid,),
      in_specs=[
          pl.BlockSpec((1, _ROWS, d), lambda i: (0, i, 0)),
          pl.BlockSpec((1, _ROWS, d), lambda i: (1, i, 0)),
          pl.BlockSpec((_ROWS, d), lambda i: (i, 0)),
          pl.BlockSpec((_ROWS, 1), lambda i: (i, 0)),
          pl.BlockSpec((1, d), lambda i: (0, 0)),
          pl.BlockSpec((d, d), lambda i: (0, 0)),
      ],
      out_specs=pl.BlockSpec((_ROWS, d), lambda i: (i, 0)),
      out_shape=jax.ShapeDtypeStruct((n, d), jnp.float32),
  )(agg, agg, y, dinv, b.reshape(1, d), w)


def _tc_final(agg, y, dinv, b, batch, wl, bl, g=64):
  n, d = y.shape
  c = wl.shape[1]
  grid = n // _ROWS
  batch_r = batch.reshape(grid, 1, _ROWS)

  def body(agg_a, agg_b, y_ref, dinv_ref, b_ref, batch_ref, wl_ref, bl_ref,
           out_ref, acc_ref):
    i = pl.program_id(0)
    dv = dinv_ref[...]
    h = jnp.maximum(
        dv * (agg_a[0] + agg_b[0] + y_ref[...]) + b_ref[...], 0.0)
    ids = batch_ref[0, 0, :]
    oh = (lax.broadcasted_iota(jnp.int32, (g, _ROWS), 0)
          == jnp.reshape(ids, (1, _ROWS))).astype(jnp.float32)
    part = jnp.dot(oh, h, preferred_element_type=jnp.float32)

    @pl.when(i == 0)
    def _():
      acc_ref[...] = part

    @pl.when(i > 0)
    def _():
      acc_ref[...] = acc_ref[...] + part

    @pl.when(i == grid - 1)
    def _():
      logits = jnp.dot(acc_ref[...], wl_ref[...],
                       preferred_element_type=jnp.float32) + bl_ref[...]
      m = jnp.max(logits, axis=1, keepdims=True)
      lse = jnp.log(jnp.sum(jnp.exp(logits - m), axis=1, keepdims=True)) + m
      out_ref[...] = logits - lse

  return pl.pallas_call(
      body,
      grid=(grid,),
      in_specs=[
          pl.BlockSpec((1, _ROWS, d), lambda i: (0, i, 0)),
          pl.BlockSpec((1, _ROWS, d), lambda i: (1, i, 0)),
          pl.BlockSpec((_ROWS, d), lambda i: (i, 0)),
          pl.BlockSpec((_ROWS, 1), lambda i: (i, 0)),
          pl.BlockSpec((1, d), lambda i: (0, 0)),
          pl.BlockSpec((1, 1, _ROWS), lambda i: (i, 0, 0)),
          pl.BlockSpec((d, c), lambda i: (0, 0)),
          pl.BlockSpec((1, c), lambda i: (0, 0)),
      ],
      out_specs=pl.BlockSpec((g, c), lambda i: (0, 0)),
      out_shape=jax.ShapeDtypeStruct((g, c), jnp.float32),
      scratch_shapes=[pltpu.VMEM((g, d), jnp.float32)],
  )(agg, agg, y, dinv, b.reshape(1, d), batch_r, wl, bl.reshape(1, c))


def kernel(x, edge_index, batch, W1, b1, W2, b2, W3, b3, Wl, bl):
  n, _ = x.shape
  npad = ((n + NS * CH - 1) // (NS * CH)) * NS * CH
  e = edge_index.shape[1]
  gr = 2 * NW * _RING * CH
  ep = ((e + gr - 1) // gr) * gr
  nrnd = ep // (NW * _RING * CH)
  src = jnp.concatenate(
      [edge_index[0], jnp.zeros((ep - e,), edge_index.dtype)])
  dst = jnp.concatenate(
      [edge_index[1], jnp.full((ep - e,), npad - 1, edge_index.dtype)])
  srcr = src.reshape(NW, nrnd, _RING, CH)
  dstr = dst.reshape(NW, nrnd, _RING, CH)

  degp = _sc_degree(dstr, npad)
  y1, dinv = _tc_first(degp, x, W1)
  p1 = _sc_aggregate(y1, srcr, dstr, npad)
  y2 = _tc_mid(p1, y1, dinv, b1, W2)
  p2 = _sc_aggregate(y2, srcr, dstr, npad)
  y3 = _tc_mid(p2, y2, dinv, b2, W3)
  p3 = _sc_aggregate(y3, srcr, dstr, npad)
  return _tc_final(p3, y3, dinv, b3, batch, Wl, bl)


# speedup vs baseline: 3.8426x; 1.7897x over previous
import functools

import jax
import jax.numpy as jnp
from jax import lax
from jax.experimental import pallas as pl
from jax.experimental.pallas import tpu as pltpu
from jax.experimental.pallas import tpu_sc as plsc

NC = 2
NS = 16
NW = NC * NS
CH = 40
DW = 16


def _sc_mesh():
  return plsc.VectorSubcoreMesh(
      core_axis_name="c", subcore_axis_name="s", num_cores=NC, num_subcores=NS)


def _sc_degree(dstr, npad):
  nrnd = dstr.shape[1]
  rpt = npad // NS
  zr = 128

  @functools.partial(
      pl.kernel,
      out_type=jax.ShapeDtypeStruct((NC, npad, DW), jnp.float32),
      mesh=_sc_mesh(),
      scratch_types=[
          pltpu.VMEM((2, _RING, CH), jnp.int32),
          pltpu.VMEM((CH, DW), jnp.float32),
          pltpu.VMEM((zr, DW), jnp.float32),
          pltpu.VMEM_SHARED((npad, DW), jnp.float32),
          pltpu.SemaphoreType.DMA,
          pltpu.SemaphoreType.DMA((2,)),
      ],
  )
  def k(dst_hbm, out_hbm, didx, ones, zbuf, acc, dsem, isem):
    c = lax.axis_index("c")
    s = lax.axis_index("s")
    w = c * NS + s

    pltpu.async_copy(dst_hbm.at[w, 0], didx.at[0], isem.at[0])
    pltpu.async_copy(dst_hbm.at[w, 1], didx.at[1], isem.at[1])

    def fill_z(r, _):
      zbuf[r, :] = jnp.zeros((DW,), jnp.float32)
      return 0
    lax.fori_loop(0, zr, fill_z, 0)

    def fill_o(r, _):
      ones[r, :] = jnp.ones((DW,), jnp.float32)
      return 0
    lax.fori_loop(0, CH, fill_o, 0)

    for z in range(rpt // zr):
      pltpu.async_copy(zbuf, acc.at[pl.ds(s * rpt + z * zr, zr)], dsem)
    for z in range(rpt // zr):
      pltpu.make_async_copy(zbuf, acc.at[pl.ds(s * rpt, zr)], dsem).wait()
    plsc.subcore_barrier()

    def do_round(m, p):
      for b in range(_RING):
        pltpu.async_copy(ones, acc.at[didx.at[p].at[b]], dsem, add=True)
      for b in range(_RING):
        pltpu.make_async_copy(ones, acc.at[didx.at[p].at[b]], dsem).wait()

    def dround(t, _):
      for p in range(2):
        m = 2 * t + p
        pltpu.make_async_copy(dst_hbm.at[w, m], didx.at[p], isem.at[p]).wait()
        do_round(m, p)

        @pl.when(m + 2 < nrnd)
        def _():
          pltpu.async_copy(dst_hbm.at[w, m + 2], didx.at[p], isem.at[p])
      return 0
    lax.fori_loop(0, nrnd // 2, dround, 0)
    if nrnd % 2:
      m = nrnd - 1
      pltpu.make_async_copy(dst_hbm.at[w, m], didx.at[0], isem.at[0]).wait()
      do_round(m, 0)
    plsc.subcore_barrier()

    pltpu.sync_copy(acc.at[pl.ds(s * rpt, rpt)],
                    out_hbm.at[c, pl.ds(s * rpt, rpt)])

  return k(dstr)


_RING = 5


def _sc_aggregate(y, srcr, dstr, npad):
  n, d = y.shape
  nrnd = srcr.shape[1]
  rpt = npad // NS
  assert rpt % CH == 0

  @functools.partial(
      pl.kernel,
      out_type=jax.ShapeDtypeStruct((NC, npad, d), jnp.float32),
      mesh=_sc_mesh(),
      scratch_types=[
          pltpu.VMEM((2, _RING, CH), jnp.int32),
          pltpu.VMEM((2, _RING, CH), jnp.int32),
          pltpu.VMEM((_RING, CH, d), jnp.float32),
          pltpu.VMEM_SHARED((npad, d), jnp.float32),
          pltpu.SemaphoreType.DMA((_RING,)),
          pltpu.SemaphoreType.DMA((2,)),
      ],
  )
  def k(y_hbm, src_hbm, dst_hbm, out_hbm, sidx, didx, rows, acc, gsem, isem):
    c = lax.axis_index("c")
    s = lax.axis_index("s")
    w = c * NS + s

    for p in range(2):
      pltpu.async_copy(src_hbm.at[w, p], sidx.at[p], isem.at[p])
      pltpu.async_copy(dst_hbm.at[w, p], didx.at[p], isem.at[p])

    def fill_z(r, _):
      for j in range(d // 16):
        rows[0, r, pl.ds(j * 16, 16)] = jnp.zeros((16,), jnp.float32)
      return 0
    lax.fori_loop(0, CH, fill_z, 0)
    for z in range(rpt // CH):
      pltpu.async_copy(rows.at[0], acc.at[pl.ds(s * rpt + z * CH, CH)],
                       gsem.at[0])
    for z in range(rpt // CH):
      pltpu.make_async_copy(rows.at[0], acc.at[pl.ds(s * rpt, CH)],
                            gsem.at[0]).wait()
    plsc.subcore_barrier()

    def wait_idx(m, p):
      pltpu.make_async_copy(src_hbm.at[w, m], sidx.at[p], isem.at[p]).wait()
      pltpu.make_async_copy(dst_hbm.at[w, m], didx.at[p], isem.at[p]).wait()

    def gather(p, b):
      pltpu.async_copy(y_hbm.at[sidx.at[p].at[b]], rows.at[b], gsem.at[b])

    def wait_gather(p, b):
      pltpu.make_async_copy(y_hbm.at[sidx.at[p].at[b]], rows.at[b],
                            gsem.at[b]).wait()

    wait_idx(0, 0)
    for b in range(_RING):
      gather(0, b)

    def dround(t, _):
      for p in range(2):
        m = 2 * t + p
        last = m + 1 >= nrnd

        @pl.when(jnp.logical_not(last))
        def _():
          wait_idx(m + 1, 1 - p)
        for b in range(_RING):
          wait_gather(p, b)
          pltpu.sync_copy(rows.at[b], acc.at[didx.at[p].at[b]], add=True)

          @pl.when(jnp.logical_not(last))
          def _():
            gather(1 - p, b)

        @pl.when(m + 2 < nrnd)
        def _():
          pltpu.async_copy(src_hbm.at[w, m + 2], sidx.at[p], isem.at[p])
          pltpu.async_copy(dst_hbm.at[w, m + 2], didx.at[p], isem.at[p])
      return 0
    lax.fori_loop(0, nrnd // 2, dround, 0)
    assert nrnd % 2 == 0
    plsc.subcore_barrier()

    pltpu.sync_copy(acc.at[pl.ds(s * rpt, rpt)],
                    out_hbm.at[c, pl.ds(s * rpt, rpt)])

  return k(y, srcr, dstr)


_ROWS = 2000


def _tc_first(degp, x, w1):
  n, d = x.shape
  grid = n // _ROWS

  def body(deg_a, deg_b, x_ref, w_ref, y_ref, dinv_ref):
    dg = 1.0 + deg_a[0, :, 0:1] + deg_b[0, :, 0:1]
    dinv = lax.rsqrt(dg)
    y_ref[...] = dinv * jnp.dot(x_ref[...], w_ref[...],
                                preferred_element_type=jnp.float32)
    dinv_ref[...] = dinv

  return pl.pallas_call(
      body,
      grid=(grid,),
      in_specs=[
          pl.BlockSpec((1, _ROWS, DW), lambda i: (0, i, 0)),
          pl.BlockSpec((1, _ROWS, DW), lambda i: (1, i, 0)),
          pl.BlockSpec((_ROWS, d), lambda i: (i, 0)),
          pl.BlockSpec((d, d), lambda i: (0, 0)),
      ],
      out_specs=[
          pl.BlockSpec((_ROWS, d), lambda i: (i, 0)),
          pl.BlockSpec((_ROWS, 1), lambda i: (i, 0)),
      ],
      out_shape=[
          jax.ShapeDtypeStruct((n, d), jnp.float32),
          jax.ShapeDtypeStruct((n, 1), jnp.float32),
      ],
  )(degp, degp, x, w1)


def _tc_mid(agg, y, dinv, b, w):
  n, d = y.shape
  grid = n // _ROWS

  def body(agg_a, agg_b, y_ref, dinv_ref, b_ref, w_ref, out_ref):
    dv = dinv_ref[...]
    h = jnp.maximum(
        dv * (agg_a[0] + agg_b[0] + y_ref[...]) + b_ref[...], 0.0)
    out_ref[...] = dv * jnp.dot(h, w_ref[...],
                                preferred_element_type=jnp.float32)

  return pl.pallas_call(
      body,
      grid=(grid,),
      in_specs=[
          pl.BlockSpec((1, _ROWS, d), lambda i: (0, i, 0)),
          pl.BlockSpec((1, _ROWS, d), lambda i: (1, i, 0)),
          pl.BlockSpec((_ROWS, d), lambda i: (i, 0)),
          pl.BlockSpec((_ROWS, 1), lambda i: (i, 0)),
          pl.BlockSpec((1, d), lambda i: (0, 0)),
          pl.BlockSpec((d, d), lambda i: (0, 0)),
      ],
      out_specs=pl.BlockSpec((_ROWS, d), lambda i: (i, 0)),
      out_shape=jax.ShapeDtypeStruct((n, d), jnp.float32),
  )(agg, agg, y, dinv, b.reshape(1, d), w)


def _tc_final(agg, y, dinv, b, batch, wl, bl, g=64):
  n, d = y.shape
  c = wl.shape[1]
  grid = n // _ROWS
  batch_r = batch.reshape(grid, 1, _ROWS)

  def body(agg_a, agg_b, y_ref, dinv_ref, b_ref, batch_ref, wl_ref, bl_ref,
           out_ref, acc_ref):
    i = pl.program_id(0)
    dv = dinv_ref[...]
    h = jnp.maximum(
        dv * (agg_a[0] + agg_b[0] + y_ref[...]) + b_ref[...], 0.0)
    ids = batch_ref[0, 0, :]
    oh = (lax.broadcasted_iota(jnp.int32, (g, _ROWS), 0)
          == jnp.reshape(ids, (1, _ROWS))).astype(jnp.float32)
    part = jnp.dot(oh, h, preferred_element_type=jnp.float32)

    @pl.when(i == 0)
    def _():
      acc_ref[...] = part

    @pl.when(i > 0)
    def _():
      acc_ref[...] = acc_ref[...] + part

    @pl.when(i == grid - 1)
    def _():
      logits = jnp.dot(acc_ref[...], wl_ref[...],
                       preferred_element_type=jnp.float32) + bl_ref[...]
      m = jnp.max(logits, axis=1, keepdims=True)
      lse = jnp.log(jnp.sum(jnp.exp(logits - m), axis=1, keepdims=True)) + m
      out_ref[...] = logits - lse

  return pl.pallas_call(
      body,
      grid=(grid,),
      in_specs=[
          pl.BlockSpec((1, _ROWS, d), lambda i: (0, i, 0)),
          pl.BlockSpec((1, _ROWS, d), lambda i: (1, i, 0)),
          pl.BlockSpec((_ROWS, d), lambda i: (i, 0)),
          pl.BlockSpec((_ROWS, 1), lambda i: (i, 0)),
          pl.BlockSpec((1, d), lambda i: (0, 0)),
          pl.BlockSpec((1, 1, _ROWS), lambda i: (i, 0, 0)),
          pl.BlockSpec((d, c), lambda i: (0, 0)),
          pl.BlockSpec((1, c), lambda i: (0, 0)),
      ],
      out_specs=pl.BlockSpec((g, c), lambda i: (0, 0)),
      out_shape=jax.ShapeDtypeStruct((g, c), jnp.float32),
      scratch_shapes=[pltpu.VMEM((g, d), jnp.float32)],
  )(agg, agg, y, dinv, b.reshape(1, d), batch_r, wl, bl.reshape(1, c))


def kernel(x, edge_index, batch, W1, b1, W2, b2, W3, b3, Wl, bl):
  n, _ = x.shape
  npad = ((n + NS * CH - 1) // (NS * CH)) * NS * CH
  e = edge_index.shape[1]
  gr = 2 * NW * _RING * CH
  ep = ((e + gr - 1) // gr) * gr
  nrnd = ep // (NW * _RING * CH)
  src = jnp.concatenate(
      [edge_index[0], jnp.zeros((ep - e,), edge_index.dtype)])
  dst = jnp.concatenate(
      [edge_index[1], jnp.full((ep - e,), npad - 1, edge_index.dtype)])
  srcr = src.reshape(NW, nrnd, _RING, CH)
  dstr = dst.reshape(NW, nrnd, _RING, CH)

  degp = _sc_degree(dstr, npad)
  y1, dinv = _tc_first(degp, x, W1)
  p1 = _sc_aggregate(y1, srcr, dstr, npad)
  y2 = _tc_mid(p1, y1, dinv, b1, W2)
  p2 = _sc_aggregate(y2, srcr, dstr, npad)
  y3 = _tc_mid(p2, y2, dinv, b2, W3)
  p3 = _sc_aggregate(y3, srcr, dstr, npad)
  return _tc_final(p3, y3, dinv, b3, batch, Wl, bl)
